# Initial kernel scaffold; baseline (speedup 1.0000x reference)
import jax
import jax.numpy as jnp
from jax.experimental import pallas as pl


def kernel(logits, temperature, top_p, top_k):
    raise NotImplementedError("write your pallas kernel here")


# speedup vs baseline: 39.9380x; 39.9380x over previous
import functools

import jax
import jax.numpy as jnp
from jax import lax
from jax.experimental import pallas as pl
from jax.experimental.pallas import tpu as pltpu
from jax.experimental.pallas import tpu_sc as plsc

W = 128
NSEL = 50
NPAD = 56
NEG = float("-inf")
BIG = 2 ** 30


def _k1_body(x_ref, bm_ref, m_ref, s_ref):
    x = x_ref[...]
    bmx = jnp.max(x, axis=2)
    m = jnp.max(bmx, axis=1, keepdims=True)
    e = jnp.exp(x - m[:, :, None])
    s = jnp.sum(jnp.sum(e, axis=2), axis=1, keepdims=True)
    bm_ref[...] = bmx
    m_ref[...] = m
    s_ref[...] = s


def _k2_body(nb, bm_ref, flat_ref, work_ref, acc_ref):
    b, _ = bm_ref.shape
    riota = lax.broadcasted_iota(jnp.int32, (b, 1), 0)
    biota = lax.broadcasted_iota(jnp.int32, (b, nb), 1)
    liota = lax.broadcasted_iota(jnp.int32, (b, NPAD), 1)
    work_ref[...] = bm_ref[...]
    acc_ref[...] = jnp.broadcast_to(riota * nb, (b, NPAD))

    def step(t, carry):
        w = work_ref[...]
        mx = jnp.max(w, axis=1, keepdims=True)
        bid = jnp.min(jnp.where(w == mx, biota, BIG), axis=1, keepdims=True)
        acc_ref[...] = jnp.where(liota == t, riota * nb + bid, acc_ref[...])
        work_ref[...] = jnp.where(biota == bid, NEG, w)
        return carry

    lax.fori_loop(0, NSEL, step, 0)
    flat_ref[...] = acc_ref[...]


def _k3_body(tbl_hbm, gtbl_hbm, flat_hbm, out_hbm, gout_hbm,
             idx_v, rows_v, grows_v, sem, gsem):
    wid = lax.axis_index("s") * 2 + lax.axis_index("c")
    base = wid * 4
    pltpu.sync_copy(flat_hbm.at[pl.ds(base, 4)], idx_v)
    copies = []
    for j in range(4):
        copies.append(pltpu.async_copy(tbl_hbm.at[idx_v.at[j]], rows_v.at[j], sem))
        copies.append(pltpu.async_copy(gtbl_hbm.at[idx_v.at[j]], grows_v.at[j], gsem))
    for c in copies:
        c.wait()
    pltpu.sync_copy(rows_v, out_hbm.at[pl.ds(base, 4)])
    pltpu.sync_copy(grows_v, gout_hbm.at[pl.ds(base, 4)])


def _k4_body(nb, cands_ref, gcands_ref, flat_ref, m_ref, s_ref, temp_ref,
             topp_ref, topk_ref, samp_ref, tkv_ref, tki_ref, slp_ref,
             work_ref, idx_ref, vals_ref, idxs_ref, gums_ref):
    b = cands_ref.shape[0]
    riota = lax.broadcasted_iota(jnp.int32, (b, 1), 0)
    siota = lax.broadcasted_iota(jnp.int32, (b, NPAD, W), 1)
    oiota = lax.broadcasted_iota(jnp.int32, (b, NPAD, W), 2)
    liota = lax.broadcasted_iota(jnp.int32, (b, 64), 1)

    bcol = flat_ref[...] - riota * nb
    idx_ref[...] = bcol[:, :, None] * W + oiota
    work_ref[...] = jnp.where(siota < NSEL, cands_ref[...], NEG)
    vals_ref[...] = jnp.full((b, 64), NEG, jnp.float32)
    idxs_ref[...] = jnp.full((b, 64), BIG, jnp.int32)
    gums_ref[...] = jnp.zeros((b, 64), jnp.float32)

    def step(t, carry):
        w = work_ref[...]
        mx = jnp.max(jnp.max(w, axis=2), axis=1, keepdims=True)
        ismax = w == mx[:, :, None]
        idx = idx_ref[...]
        chid = jnp.min(jnp.min(jnp.where(ismax, idx, BIG), axis=2),
                       axis=1, keepdims=True)
        chosen = ismax & (idx == chid[:, :, None])
        gum = jnp.sum(jnp.sum(jnp.where(chosen, gcands_ref[...], 0.0), axis=2),
                      axis=1, keepdims=True)
        sel = liota == t
        vals_ref[...] = jnp.where(sel, mx, vals_ref[...])
        idxs_ref[...] = jnp.where(sel, chid, idxs_ref[...])
        gums_ref[...] = jnp.where(sel, gum, gums_ref[...])
        work_ref[...] = jnp.where(chosen, NEG, w)
        return carry

    lax.fori_loop(0, NSEL, step, 0)

    vals = vals_ref[...]
    idxs = idxs_ref[...]
    gums = gums_ref[...]
    temp = temp_ref[...]
    x = vals / temp
    kth = jnp.max(jnp.where(liota == topk_ref[...] - 1, x, NEG),
                  axis=1, keepdims=True)
    keep1 = ~(x < kth)
    xm = jnp.where(keep1, x, NEG)
    p = jnp.exp(xm - x[:, 0:1])
    pn = p / jnp.sum(p, axis=1, keepdims=True)
    c = pn
    for sh in (1, 2, 4, 8, 16, 32):
        z = jnp.zeros((b, sh), jnp.float32)
        c = c + jnp.concatenate([c[:, sh:], z], axis=1)
    keep2 = (c > 1.0 - topp_ref[...]) | (liota == 0)
    keep = keep1 & keep2
    y = jnp.where(keep, xm, NEG)
    z = y + gums
    zmx = jnp.max(z, axis=1, keepdims=True)
    rand = jnp.min(jnp.where(z == zmx, idxs, BIG), axis=1, keepdims=True)
    greedy = idxs[:, 0:1]
    sampled = jnp.where(temp < 1e-5, greedy, rand)
    vs = jnp.max(jnp.where(idxs == sampled, vals, NEG), axis=1, keepdims=True)
    lse = m_ref[...] + jnp.log(s_ref[...])
    samp_ref[...] = sampled
    slp_ref[...] = vs - lse
    tkv_ref[...] = vals[:, :20] - lse
    tki_ref[...] = idxs[:, :20]


def kernel(logits, temperature, top_p, top_k):
    b, v = logits.shape
    nb = -(-v // W)
    vp = nb * W
    logits = logits.astype(jnp.float32)
    g = jax.random.gumbel(jax.random.key(42), (b, v), dtype=jnp.float32)
    if vp != v:
        logits = jnp.pad(logits, ((0, 0), (0, vp - v)), constant_values=NEG)
        g = jnp.pad(g, ((0, 0), (0, vp - v)))

    logits3 = logits.reshape(b, nb, W)
    bm, m, s = pl.pallas_call(
        _k1_body,
        grid=(b // 8,),
        in_specs=[pl.BlockSpec((8, nb, W), lambda r: (r, 0, 0))],
        out_specs=[
            pl.BlockSpec((8, nb), lambda r: (r, 0)),
            pl.BlockSpec((8, 1), lambda r: (r, 0)),
            pl.BlockSpec((8, 1), lambda r: (r, 0)),
        ],
        out_shape=[
            jax.ShapeDtypeStruct((b, nb), jnp.float32),
            jax.ShapeDtypeStruct((b, 1), jnp.float32),
            jax.ShapeDtypeStruct((b, 1), jnp.float32),
        ],
    )(logits3)

    flat = pl.pallas_call(
        functools.partial(_k2_body, nb),
        out_shape=jax.ShapeDtypeStruct((b, NPAD), jnp.int32),
        scratch_shapes=[
            pltpu.VMEM((b, nb), jnp.float32),
            pltpu.VMEM((b, NPAD), jnp.int32),
        ],
    )(bm)

    tbl = logits.reshape(b * nb, W)
    gtbl = g.reshape(b * nb, W)
    gather = pl.kernel(
        _k3_body,
        out_type=[
            jax.ShapeDtypeStruct((b, NPAD, W), jnp.float32),
            jax.ShapeDtypeStruct((b, NPAD, W), jnp.float32),
        ],
        mesh=plsc.VectorSubcoreMesh(core_axis_name="c", subcore_axis_name="s"),
        scratch_types=[
            pltpu.VMEM((4, NPAD), jnp.int32),
            pltpu.VMEM((4, NPAD, W), jnp.float32),
            pltpu.VMEM((4, NPAD, W), jnp.float32),
            pltpu.SemaphoreType.DMA,
            pltpu.SemaphoreType.DMA,
        ],
    )
    cands, gcands = gather(tbl, gtbl, flat)

    samp, tkv, tki, slp = pl.pallas_call(
        functools.partial(_k4_body, nb),
        out_shape=[
            jax.ShapeDtypeStruct((b, 1), jnp.int32),
            jax.ShapeDtypeStruct((b, 20), jnp.float32),
            jax.ShapeDtypeStruct((b, 20), jnp.int32),
            jax.ShapeDtypeStruct((b, 1), jnp.float32),
        ],
        scratch_shapes=[
            pltpu.VMEM((b, NPAD, W), jnp.float32),
            pltpu.VMEM((b, NPAD, W), jnp.int32),
            pltpu.VMEM((b, 64), jnp.float32),
            pltpu.VMEM((b, 64), jnp.int32),
            pltpu.VMEM((b, 64), jnp.float32),
        ],
    )(
        cands, gcands, flat, m, s,
        temperature.astype(jnp.float32).reshape(b, 1),
        top_p.astype(jnp.float32).reshape(b, 1),
        top_k.astype(jnp.int32).reshape(b, 1),
    )

    return samp, tkv, tki.astype(jnp.int64), slp


# speedup vs baseline: 67.2457x; 1.6838x over previous
import functools

import jax
import jax.numpy as jnp
from jax import lax
from jax.experimental import pallas as pl
from jax.experimental.pallas import tpu as pltpu
from jax.experimental.pallas import tpu_sc as plsc

W = 128
NSEL = 50
NPAD = 56
NEG = float("-inf")
BIG = 2 ** 30


def _k1_body(x_ref, bm_ref, m_ref, s_ref):
    x = x_ref[...]
    bmx = jnp.max(x, axis=2)
    m = jnp.max(bmx, axis=1, keepdims=True)
    e = jnp.exp(x - m[:, :, None])
    s = jnp.sum(jnp.sum(e, axis=2), axis=1, keepdims=True)
    bm_ref[...] = bmx
    m_ref[...] = m
    s_ref[...] = s


def _k2_body(nb, bm_ref, flat_ref, work_ref, acc_ref):
    b, _ = bm_ref.shape
    riota = lax.broadcasted_iota(jnp.int32, (b, 1), 0)
    biota = lax.broadcasted_iota(jnp.int32, (b, nb), 1)
    liota = lax.broadcasted_iota(jnp.int32, (b, NPAD), 1)
    work_ref[...] = bm_ref[...]
    acc_ref[...] = jnp.broadcast_to(riota * nb, (b, NPAD))

    def step(t, carry):
        w = work_ref[...]
        mx = jnp.max(w, axis=1, keepdims=True)
        bid = jnp.min(jnp.where(w == mx, biota, BIG), axis=1, keepdims=True)
        acc_ref[...] = jnp.where(liota == t, riota * nb + bid, acc_ref[...])
        work_ref[...] = jnp.where(biota == bid, NEG, w)
        return carry

    lax.fori_loop(0, NSEL, step, 0)
    flat_ref[...] = acc_ref[...]


def _k3_body(tbl_hbm, flat_hbm, out_hbm, idx_v, rows_v, sem):
    wid = lax.axis_index("s") * 2 + lax.axis_index("c")
    base = wid * 4
    pltpu.sync_copy(flat_hbm.at[pl.ds(base, 4)], idx_v)
    copies = []
    for j in range(4):
        copies.append(pltpu.async_copy(tbl_hbm.at[idx_v.at[j]], rows_v.at[j], sem))
    for c in copies:
        c.wait()
    pltpu.sync_copy(rows_v, out_hbm.at[pl.ds(base, 4)])


def _tf_gumbel(flat):
    rots = ((13, 15, 26, 6), (17, 29, 16, 24))
    k0 = jnp.int32(0)
    k1 = jnp.int32(42)
    ks = (k0, k1, k0 ^ k1 ^ jnp.int32(0x1BD11BDA))
    x0 = jnp.zeros_like(flat) + ks[0]
    x1 = flat + ks[1]
    for d in range(5):
        for r in rots[d % 2]:
            x0 = x0 + x1
            x1 = (x1 << r) | lax.shift_right_logical(x1, 32 - r)
            x1 = x0 ^ x1
        x0 = x0 + ks[(d + 1) % 3]
        x1 = x1 + ks[(d + 2) % 3] + jnp.int32(d + 1)
    bits = x0 ^ x1
    fl = lax.bitcast_convert_type(
        lax.shift_right_logical(bits, 9) | jnp.int32(0x3F800000), jnp.float32)
    fl = fl - jnp.float32(1.0)
    tiny = jnp.float32(jnp.finfo(jnp.float32).tiny)
    u = jnp.maximum(tiny, fl * (jnp.float32(1.0) - tiny) + tiny)
    return -jnp.log(-jnp.log(u))


def _k4_body(nb, v, cands_ref, flat_ref, m_ref, s_ref, temp_ref,
             topp_ref, topk_ref, samp_ref, tkv_ref, tki_ref, slp_ref,
             work_ref, idx_ref, vals_ref, idxs_ref):
    b = cands_ref.shape[0]
    riota = lax.broadcasted_iota(jnp.int32, (b, 1), 0)
    siota = lax.broadcasted_iota(jnp.int32, (b, NPAD, W), 1)
    oiota = lax.broadcasted_iota(jnp.int32, (b, NPAD, W), 2)
    liota = lax.broadcasted_iota(jnp.int32, (b, 64), 1)

    bcol = flat_ref[...] - riota * nb
    idx_ref[...] = bcol[:, :, None] * W + oiota
    work_ref[...] = jnp.where(siota < NSEL, cands_ref[...], NEG)
    vals_ref[...] = jnp.full((b, 64), NEG, jnp.float32)
    idxs_ref[...] = jnp.full((b, 64), BIG, jnp.int32)

    def step(t, carry):
        w = work_ref[...]
        mx = jnp.max(jnp.max(w, axis=2), axis=1, keepdims=True)
        ismax = w == mx[:, :, None]
        idx = idx_ref[...]
        chid = jnp.min(jnp.min(jnp.where(ismax, idx, BIG), axis=2),
                       axis=1, keepdims=True)
        chosen = ismax & (idx == chid[:, :, None])
        sel = liota == t
        vals_ref[...] = jnp.where(sel, mx, vals_ref[...])
        idxs_ref[...] = jnp.where(sel, chid, idxs_ref[...])
        work_ref[...] = jnp.where(chosen, NEG, w)
        return carry

    lax.fori_loop(0, NSEL, step, 0)

    vals = vals_ref[...]
    idxs = idxs_ref[...]
    gums = _tf_gumbel(riota * v + idxs)
    temp = temp_ref[...]
    x = vals / temp
    kth = jnp.max(jnp.where(liota == topk_ref[...] - 1, x, NEG),
                  axis=1, keepdims=True)
    keep1 = ~(x < kth)
    xm = jnp.where(keep1, x, NEG)
    p = jnp.exp(xm - x[:, 0:1])
    pn = p / jnp.sum(p, axis=1, keepdims=True)
    c = pn
    for sh in (1, 2, 4, 8, 16, 32):
        z = jnp.zeros((b, sh), jnp.float32)
        c = c + jnp.concatenate([c[:, sh:], z], axis=1)
    keep2 = (c > 1.0 - topp_ref[...]) | (liota == 0)
    keep = keep1 & keep2
    y = jnp.where(keep, xm, NEG)
    z = y + gums
    zmx = jnp.max(z, axis=1, keepdims=True)
    rand = jnp.min(jnp.where(z == zmx, idxs, BIG), axis=1, keepdims=True)
    greedy = idxs[:, 0:1]
    sampled = jnp.where(temp < 1e-5, greedy, rand)
    vs = jnp.max(jnp.where(idxs == sampled, vals, NEG), axis=1, keepdims=True)
    lse = m_ref[...] + jnp.log(s_ref[...])
    samp_ref[...] = sampled
    slp_ref[...] = vs - lse
    tkv_ref[...] = vals[:, :20] - lse
    tki_ref[...] = idxs[:, :20]


def kernel(logits, temperature, top_p, top_k):
    b, v = logits.shape
    nb = -(-v // W)
    vp = nb * W
    logits = logits.astype(jnp.float32)
    if vp != v:
        logits = jnp.pad(logits, ((0, 0), (0, vp - v)), constant_values=NEG)

    logits3 = logits.reshape(b, nb, W)
    bm, m, s = pl.pallas_call(
        _k1_body,
        grid=(b // 8,),
        in_specs=[pl.BlockSpec((8, nb, W), lambda r: (r, 0, 0))],
        out_specs=[
            pl.BlockSpec((8, nb), lambda r: (r, 0)),
            pl.BlockSpec((8, 1), lambda r: (r, 0)),
            pl.BlockSpec((8, 1), lambda r: (r, 0)),
        ],
        out_shape=[
            jax.ShapeDtypeStruct((b, nb), jnp.float32),
            jax.ShapeDtypeStruct((b, 1), jnp.float32),
            jax.ShapeDtypeStruct((b, 1), jnp.float32),
        ],
    )(logits3)

    flat = pl.pallas_call(
        functools.partial(_k2_body, nb),
        out_shape=jax.ShapeDtypeStruct((b, NPAD), jnp.int32),
        scratch_shapes=[
            pltpu.VMEM((b, nb), jnp.float32),
            pltpu.VMEM((b, NPAD), jnp.int32),
        ],
    )(bm)

    tbl = logits.reshape(b * nb, W)
    gather = pl.kernel(
        _k3_body,
        out_type=[jax.ShapeDtypeStruct((b, NPAD, W), jnp.float32)],
        mesh=plsc.VectorSubcoreMesh(core_axis_name="c", subcore_axis_name="s"),
        scratch_types=[
            pltpu.VMEM((4, NPAD), jnp.int32),
            pltpu.VMEM((4, NPAD, W), jnp.float32),
            pltpu.SemaphoreType.DMA,
        ],
    )
    (cands,) = gather(tbl, flat)

    samp, tkv, tki, slp = pl.pallas_call(
        functools.partial(_k4_body, nb, v),
        out_shape=[
            jax.ShapeDtypeStruct((b, 1), jnp.int32),
            jax.ShapeDtypeStruct((b, 20), jnp.float32),
            jax.ShapeDtypeStruct((b, 20), jnp.int32),
            jax.ShapeDtypeStruct((b, 1), jnp.float32),
        ],
        scratch_shapes=[
            pltpu.VMEM((b, NPAD, W), jnp.float32),
            pltpu.VMEM((b, NPAD, W), jnp.int32),
            pltpu.VMEM((b, 64), jnp.float32),
            pltpu.VMEM((b, 64), jnp.int32),
        ],
    )(
        cands, flat, m, s,
        temperature.astype(jnp.float32).reshape(b, 1),
        top_p.astype(jnp.float32).reshape(b, 1),
        top_k.astype(jnp.int32).reshape(b, 1),
    )

    return samp, tkv, tki.astype(jnp.int64), slp
